# baseline (speedup 1.0000x reference)
import jax
import jax.numpy as jnp
from jax import lax
from jax.experimental import pallas as pl
from jax.experimental.pallas import tpu as pltpu
from jax.experimental.pallas import tpu_sc as plsc

N = 10000
D = 128
H = 16
C = 40
NC = 2
NS = 16
NW = NC * NS
CH = 125
NB = 4
NK = 80
NT = 10240
ZR = NT // NS
ZR2 = NT // (NC * NS)
P8 = NT // 8

_mesh = plsc.VectorSubcoreMesh(
    core_axis_name="c", subcore_axis_name="s", num_cores=NC, num_subcores=NS)


def _newton_rsqrt(d):
    i = plsc.bitcast(d, jnp.int32)
    i = jnp.int32(0x5F3759DF) - lax.shift_right_logical(i, 1)
    y = plsc.bitcast(i, jnp.float32)
    half_d = 0.5 * d
    for _ in range(3):
        y = y * (1.5 - half_d * y * y)
    return y


def _sc_disb_body(dst_hbm, out_hbm, idx_d, ones_v, stage_v, pk_v, dacc):
    c = lax.axis_index("c")
    s = lax.axis_index("s")

    def _fill(i, carry):
        stage_v[pl.ds(i * 16, 16)] = jnp.zeros((16,), jnp.float32)
        return carry
    lax.fori_loop(0, ZR // 16, _fill, 0)
    pltpu.sync_copy(stage_v, dacc.at[pl.ds(s * ZR, ZR)])

    def _ones(i, carry):
        ones_v[pl.ds(i * 16, 16)] = jnp.ones((16,), jnp.float32)
        return carry
    lax.fori_loop(0, 128 // 16, _ones, 0)
    pltpu.sync_copy(dst_hbm.at[2 * s], idx_d.at[pl.ds(0, NK)])
    pltpu.sync_copy(dst_hbm.at[2 * s + 1], idx_d.at[pl.ds(NK, NK)])
    plsc.subcore_barrier()

    def _chunk(j, carry):
        pltpu.sync_copy(ones_v.at[pl.ds(0, CH)], dacc.at[idx_d.at[j]],
                        add=True)
        return carry
    lax.fori_loop(0, 2 * NK, _chunk, 0)
    plsc.subcore_barrier()

    base = pl.multiple_of(c * (NT // 2) + s * ZR2, 8)
    pltpu.sync_copy(dacc.at[pl.ds(base, ZR2)], stage_v.at[pl.ds(0, ZR2)])

    def _rows(v, carry):
        d = stage_v[pl.ds(v * 16, 16)] + 1.0
        y = _newton_rsqrt(d)
        for l in range(16):
            pk_v[v * 16 + l, :] = jnp.broadcast_to(y[l], (16,))
        return carry
    lax.fori_loop(0, ZR2 // 16, _rows, 0)
    pltpu.sync_copy(pk_v, out_hbm.at[pl.ds(base, ZR2)])


_sc_disb_scratch = [
    pltpu.VMEM((2 * NK, CH), jnp.int32),
    pltpu.VMEM((128,), jnp.float32),
    pltpu.VMEM((ZR,), jnp.float32),
    pltpu.VMEM((ZR2, H), jnp.float32),
    pltpu.VMEM_SHARED((NT,), jnp.float32),
]

_sc_disb = pl.kernel(
    _sc_disb_body,
    out_type=jax.ShapeDtypeStruct((NT, H), jnp.float32),
    mesh=_mesh,
    scratch_types=_sc_disb_scratch,
    compiler_params=pltpu.CompilerParams(use_tc_tiling_on_sc=False, needs_layout_passes=False),
)


def _sc_scatter_body(q_hbm, src_hbm, dst_hbm, out0, out1,
                     idx_s, idx_d, rows, stage_v, zacc, sems):
    c = lax.axis_index("c")
    s = lax.axis_index("s")
    wid = s * NC + c

    def _fill(i, carry):
        stage_v[i, :] = jnp.zeros((16,), jnp.float32)
        return carry
    lax.fori_loop(0, ZR, _fill, 0)
    pltpu.sync_copy(stage_v, zacc.at[pl.ds(s * ZR, ZR)])
    pltpu.sync_copy(src_hbm.at[wid], idx_s)
    pltpu.sync_copy(dst_hbm.at[wid], idx_d)
    plsc.subcore_barrier()

    def _group(g, carry):
        j0 = g * NB
        descs = [
            pltpu.async_copy(q_hbm.at[idx_s.at[j0 + b]], rows.at[b],
                             sems.at[b])
            for b in range(NB)
        ]
        for b in range(NB):
            descs[b].wait()
            pltpu.sync_copy(rows.at[b], zacc.at[idx_d.at[j0 + b]], add=True)
        return carry
    lax.fori_loop(0, NK // NB, _group, 0)
    plsc.subcore_barrier()

    pltpu.sync_copy(zacc.at[pl.ds(s * ZR, ZR)], stage_v)

    @pl.when(c == 0)
    def _():
        pltpu.sync_copy(stage_v, out0.at[pl.ds(s * ZR, ZR)])

    @pl.when(c == 1)
    def _():
        pltpu.sync_copy(stage_v, out1.at[pl.ds(s * ZR, ZR)])


_sc_scatter_scratch = [
    pltpu.VMEM((NK, CH), jnp.int32),
    pltpu.VMEM((NK, CH), jnp.int32),
    pltpu.VMEM((NB, CH, H), jnp.float32),
    pltpu.VMEM((ZR, H), jnp.float32),
    pltpu.VMEM_SHARED((NT, H), jnp.float32),
    pltpu.SemaphoreType.DMA((NB,)),
]

_sc_scatter = pl.kernel(
    _sc_scatter_body,
    out_type=(jax.ShapeDtypeStruct((NT, H), jnp.float32),
              jax.ShapeDtypeStruct((NT, H), jnp.float32)),
    mesh=_mesh,
    scratch_types=_sc_scatter_scratch,
    compiler_params=pltpu.CompilerParams(use_tc_tiling_on_sc=False, needs_layout_passes=False),
)


RX = 1280
PX = RX // 8
RB8 = P8 // 8


def _tc0_body(x_ref, w_ref, disb_ref, q_ref):
    acc = jnp.zeros((PX, 128), jnp.float32)
    for n in range(8):
        acc += jnp.dot(x_ref[:, n, :], w_ref[n],
                       preferred_element_type=jnp.float32)
    q_ref[...] = acc * disb_ref[...]


_tc0 = pl.pallas_call(
    _tc0_body,
    grid=(NT // RX,),
    in_specs=[
        pl.BlockSpec((PX, 8, D), lambda i: (i, 0, 0)),
        pl.BlockSpec((8, D, 128), lambda i: (0, 0, 0)),
        pl.BlockSpec((PX, 128), lambda i: (i, 0)),
    ],
    out_specs=pl.BlockSpec((PX, 128), lambda i: (i, 0)),
    out_shape=jax.ShapeDtypeStruct((P8, 128), jnp.float32),
)


def _tc2_body(z0_ref, z1_ref, q1_ref, disb_ref, b_ref, q2_ref):
    z = z0_ref[...] + z1_ref[...] + q1_ref[...]
    y = jnp.maximum(disb_ref[...] * z + b_ref[...], 0.0)
    q2_ref[...] = disb_ref[...] * y


_tc2 = pl.pallas_call(
    _tc2_body,
    grid=(P8 // RB8,),
    in_specs=[
        pl.BlockSpec((RB8, 128), lambda i: (i, 0)),
        pl.BlockSpec((RB8, 128), lambda i: (i, 0)),
        pl.BlockSpec((RB8, 128), lambda i: (i, 0)),
        pl.BlockSpec((RB8, 128), lambda i: (i, 0)),
        pl.BlockSpec((1, 128), lambda i: (0, 0)),
    ],
    out_specs=pl.BlockSpec((RB8, 128), lambda i: (i, 0)),
    out_shape=jax.ShapeDtypeStruct((P8, 128), jnp.float32),
)


def _tc3_body(z0_ref, z1_ref, q2_ref, disb_ref, w_ref, b_ref, o_ref):
    u = disb_ref[...] * (z0_ref[...] + z1_ref[...] + q2_ref[...])
    v = jnp.dot(u, w_ref[...],
                preferred_element_type=jnp.float32) + b_ref[...]
    m = jnp.max(v, axis=1, keepdims=True)
    e = jnp.exp(v - m)
    lse = jnp.log(jnp.sum(e, axis=1, keepdims=True))
    o_ref[...] = v - m - lse


_tc3 = pl.pallas_call(
    _tc3_body,
    grid=(NT // RX,),
    in_specs=[
        pl.BlockSpec((RX, H), lambda i: (i, 0)),
        pl.BlockSpec((RX, H), lambda i: (i, 0)),
        pl.BlockSpec((RX, H), lambda i: (i, 0)),
        pl.BlockSpec((RX, H), lambda i: (i, 0)),
        pl.BlockSpec((H, C), lambda i: (0, 0)),
        pl.BlockSpec((1, C), lambda i: (0, 0)),
    ],
    out_specs=pl.BlockSpec((RX, C), lambda i: (i, 0)),
    out_shape=jax.ShapeDtypeStruct((N, C), jnp.float32),
)


def kernel(x, edge_index, W1, b1, W2, b2):
    srcp = edge_index[0].reshape(NW, NK, CH)
    dstp = edge_index[1].reshape(NW, NK, CH)

    disb_l = _sc_disb(dstp)
    disb = disb_l.reshape(P8, 128)
    x3 = x.reshape(N // 8, 8, D)
    w1e = jnp.kron(jnp.eye(8, dtype=x.dtype), W1).reshape(8, D, 128)
    q1 = _tc0(x3, w1e, disb)
    z0, z1 = _sc_scatter(q1.reshape(NT, H), srcp, dstp)
    q2 = _tc2(z0.reshape(P8, 128), z1.reshape(P8, 128), q1, disb,
              jnp.tile(b1, 8).reshape(1, 128))
    z0b, z1b = _sc_scatter(q2.reshape(NT, H), srcp, dstp)
    out = _tc3(z0b.reshape(NT, H), z1b.reshape(NT, H), q2.reshape(NT, H),
               disb_l, W2, b2.reshape(1, C))
    return out


# speedup vs baseline: 1.0167x; 1.0167x over previous
import jax
import jax.numpy as jnp
from jax import lax
from jax.experimental import pallas as pl
from jax.experimental.pallas import tpu as pltpu
from jax.experimental.pallas import tpu_sc as plsc

N = 10000
D = 128
H = 16
C = 40
NC = 2
NS = 16
NW = NC * NS
CH = 125
NB = 4
NK = 80
NT = 10240
ZR = NT // NS
ZR2 = NT // (NC * NS)
P8 = NT // 8

_mesh = plsc.VectorSubcoreMesh(
    core_axis_name="c", subcore_axis_name="s", num_cores=NC, num_subcores=NS)


def _newton_rsqrt(d):
    i = plsc.bitcast(d, jnp.int32)
    i = jnp.int32(0x5F3759DF) - lax.shift_right_logical(i, 1)
    y = plsc.bitcast(i, jnp.float32)
    half_d = 0.5 * d
    for _ in range(3):
        y = y * (1.5 - half_d * y * y)
    return y


def _sc_disb_body(dst_hbm, out_hbm, idx_d, ones_v, stage_v, pk_v, dacc):
    c = lax.axis_index("c")
    s = lax.axis_index("s")

    def _fill(i, carry):
        stage_v[pl.ds(i * 16, 16)] = jnp.zeros((16,), jnp.float32)
        return carry
    lax.fori_loop(0, ZR // 16, _fill, 0)
    pltpu.sync_copy(stage_v, dacc.at[pl.ds(s * ZR, ZR)])

    def _ones(i, carry):
        ones_v[pl.ds(i * 16, 16)] = jnp.ones((16,), jnp.float32)
        return carry
    lax.fori_loop(0, 128 // 16, _ones, 0)
    pltpu.sync_copy(dst_hbm.at[2 * s], idx_d.at[pl.ds(0, NK)])
    pltpu.sync_copy(dst_hbm.at[2 * s + 1], idx_d.at[pl.ds(NK, NK)])
    plsc.subcore_barrier()

    def _chunk(j, carry):
        pltpu.sync_copy(ones_v.at[pl.ds(0, CH)], dacc.at[idx_d.at[j]],
                        add=True)
        return carry
    lax.fori_loop(0, 2 * NK, _chunk, 0)
    plsc.subcore_barrier()

    base = pl.multiple_of(c * (NT // 2) + s * ZR2, 8)
    pltpu.sync_copy(dacc.at[pl.ds(base, ZR2)], stage_v.at[pl.ds(0, ZR2)])

    def _rows(v, carry):
        d = stage_v[pl.ds(v * 16, 16)] + 1.0
        y = _newton_rsqrt(d)
        for l in range(16):
            pk_v[v * 16 + l, :] = jnp.broadcast_to(y[l], (16,))
        return carry
    lax.fori_loop(0, ZR2 // 16, _rows, 0)
    pltpu.sync_copy(pk_v, out_hbm.at[pl.ds(base, ZR2)])


_sc_disb_scratch = [
    pltpu.VMEM((2 * NK, CH), jnp.int32),
    pltpu.VMEM((128,), jnp.float32),
    pltpu.VMEM((ZR,), jnp.float32),
    pltpu.VMEM((ZR2, H), jnp.float32),
    pltpu.VMEM_SHARED((NT,), jnp.float32),
]

_sc_disb = pl.kernel(
    _sc_disb_body,
    out_type=jax.ShapeDtypeStruct((NT, H), jnp.float32),
    mesh=_mesh,
    scratch_types=_sc_disb_scratch,
    compiler_params=pltpu.CompilerParams(use_tc_tiling_on_sc=False, needs_layout_passes=False),
)


def _sc_scatter_body(q_hbm, src_hbm, dst_hbm, out0, out1,
                     idx_s, idx_d, rows, stage_v, zacc, sems):
    c = lax.axis_index("c")
    s = lax.axis_index("s")
    wid = s * NC + c

    def _fill(i, carry):
        stage_v[i, :] = jnp.zeros((16,), jnp.float32)
        return carry
    lax.fori_loop(0, ZR, _fill, 0)
    pltpu.sync_copy(stage_v, zacc.at[pl.ds(s * ZR, ZR)])
    pltpu.sync_copy(src_hbm.at[wid], idx_s)
    pltpu.sync_copy(dst_hbm.at[wid], idx_d)
    plsc.subcore_barrier()

    def _group(g, carry):
        j0 = g * NB
        descs = [
            pltpu.async_copy(q_hbm.at[idx_s.at[j0 + b]], rows.at[b],
                             sems.at[b])
            for b in range(NB)
        ]
        for b in range(NB):
            descs[b].wait()
            pltpu.sync_copy(rows.at[b], zacc.at[idx_d.at[j0 + b]], add=True)
        return carry
    lax.fori_loop(0, NK // NB, _group, 0)
    plsc.subcore_barrier()

    pltpu.sync_copy(zacc.at[pl.ds(s * ZR, ZR)], stage_v)

    @pl.when(c == 0)
    def _():
        pltpu.sync_copy(stage_v, out0.at[pl.ds(s * ZR, ZR)])

    @pl.when(c == 1)
    def _():
        pltpu.sync_copy(stage_v, out1.at[pl.ds(s * ZR, ZR)])


_sc_scatter_scratch = [
    pltpu.VMEM((NK, CH), jnp.int32),
    pltpu.VMEM((NK, CH), jnp.int32),
    pltpu.VMEM((NB, CH, H), jnp.float32),
    pltpu.VMEM((ZR, H), jnp.float32),
    pltpu.VMEM_SHARED((NT, H), jnp.float32),
    pltpu.SemaphoreType.DMA((NB,)),
]

_sc_scatter = pl.kernel(
    _sc_scatter_body,
    out_type=(jax.ShapeDtypeStruct((NT, H), jnp.float32),
              jax.ShapeDtypeStruct((NT, H), jnp.float32)),
    mesh=_mesh,
    scratch_types=_sc_scatter_scratch,
    compiler_params=pltpu.CompilerParams(use_tc_tiling_on_sc=False, needs_layout_passes=False),
)


RX = 1280
PX = RX // 8
RB8 = P8 // 8


def _tc0_body(x_ref, w_ref, p_ref):
    acc = jnp.zeros((PX, 128), jnp.float32)
    for n in range(8):
        acc += jnp.dot(x_ref[:, n, :], w_ref[n],
                       preferred_element_type=jnp.float32)
    p_ref[...] = acc


_tc0 = pl.pallas_call(
    _tc0_body,
    grid=(NT // RX,),
    in_specs=[
        pl.BlockSpec((PX, 8, D), lambda i: (i, 0, 0)),
        pl.BlockSpec((8, D, 128), lambda i: (0, 0, 0)),
    ],
    out_specs=pl.BlockSpec((PX, 128), lambda i: (i, 0)),
    out_shape=jax.ShapeDtypeStruct((P8, 128), jnp.float32),
)


def _tc1_body(p_ref, disb_ref, q_ref):
    q_ref[...] = p_ref[...] * disb_ref[...]


_tc1 = pl.pallas_call(
    _tc1_body,
    grid=(P8 // RB8,),
    in_specs=[
        pl.BlockSpec((RB8, 128), lambda i: (i, 0)),
        pl.BlockSpec((RB8, 128), lambda i: (i, 0)),
    ],
    out_specs=pl.BlockSpec((RB8, 128), lambda i: (i, 0)),
    out_shape=jax.ShapeDtypeStruct((P8, 128), jnp.float32),
)


def _tc2_body(z0_ref, z1_ref, q1_ref, disb_ref, b_ref, q2_ref):
    z = z0_ref[...] + z1_ref[...] + q1_ref[...]
    y = jnp.maximum(disb_ref[...] * z + b_ref[...], 0.0)
    q2_ref[...] = disb_ref[...] * y


_tc2 = pl.pallas_call(
    _tc2_body,
    grid=(P8 // RB8,),
    in_specs=[
        pl.BlockSpec((RB8, 128), lambda i: (i, 0)),
        pl.BlockSpec((RB8, 128), lambda i: (i, 0)),
        pl.BlockSpec((RB8, 128), lambda i: (i, 0)),
        pl.BlockSpec((RB8, 128), lambda i: (i, 0)),
        pl.BlockSpec((1, 128), lambda i: (0, 0)),
    ],
    out_specs=pl.BlockSpec((RB8, 128), lambda i: (i, 0)),
    out_shape=jax.ShapeDtypeStruct((P8, 128), jnp.float32),
)


def _tc3_body(z0_ref, z1_ref, q2_ref, disb_ref, w_ref, b_ref, o_ref):
    u = disb_ref[...] * (z0_ref[...] + z1_ref[...] + q2_ref[...])
    v = jnp.dot(u, w_ref[...],
                preferred_element_type=jnp.float32) + b_ref[...]
    m = jnp.max(v, axis=1, keepdims=True)
    e = jnp.exp(v - m)
    lse = jnp.log(jnp.sum(e, axis=1, keepdims=True))
    o_ref[...] = v - m - lse


_tc3 = pl.pallas_call(
    _tc3_body,
    grid=(NT // RX,),
    in_specs=[
        pl.BlockSpec((RX, H), lambda i: (i, 0)),
        pl.BlockSpec((RX, H), lambda i: (i, 0)),
        pl.BlockSpec((RX, H), lambda i: (i, 0)),
        pl.BlockSpec((RX, H), lambda i: (i, 0)),
        pl.BlockSpec((H, C), lambda i: (0, 0)),
        pl.BlockSpec((1, C), lambda i: (0, 0)),
    ],
    out_specs=pl.BlockSpec((RX, C), lambda i: (i, 0)),
    out_shape=jax.ShapeDtypeStruct((N, C), jnp.float32),
)


def kernel(x, edge_index, W1, b1, W2, b2):
    srcp = edge_index[0].reshape(NW, NK, CH)
    dstp = edge_index[1].reshape(NW, NK, CH)

    disb_l = _sc_disb(dstp)
    disb = disb_l.reshape(P8, 128)
    x3 = x.reshape(N // 8, 8, D)
    w1e = jnp.kron(jnp.eye(8, dtype=x.dtype), W1).reshape(8, D, 128)
    p = _tc0(x3, w1e)
    q1 = _tc1(p, disb)
    z0, z1 = _sc_scatter(q1.reshape(NT, H), srcp, dstp)
    q2 = _tc2(z0.reshape(P8, 128), z1.reshape(P8, 128), q1, disb,
              jnp.tile(b1, 8).reshape(1, 128))
    z0b, z1b = _sc_scatter(q2.reshape(NT, H), srcp, dstp)
    out = _tc3(z0b.reshape(NT, H), z1b.reshape(NT, H), q2.reshape(NT, H),
               disb_l, W2, b2.reshape(1, C))
    return out


# speedup vs baseline: 1.2032x; 1.1833x over previous
import jax
import jax.numpy as jnp
from jax import lax
from jax.experimental import pallas as pl
from jax.experimental.pallas import tpu as pltpu
from jax.experimental.pallas import tpu_sc as plsc

N = 10000
D = 128
H = 16
C = 40
NC = 2
NS = 16
NW = NC * NS
CH = 125
NB = 4
NK = 80
NT = 10240
ZR = NT // NS
ZR2 = NT // (NC * NS)
P8 = NT // 8

_mesh = plsc.VectorSubcoreMesh(
    core_axis_name="c", subcore_axis_name="s", num_cores=NC, num_subcores=NS)


def _newton_rsqrt(d):
    i = plsc.bitcast(d, jnp.int32)
    i = jnp.int32(0x5F3759DF) - lax.shift_right_logical(i, 1)
    y = plsc.bitcast(i, jnp.float32)
    half_d = 0.5 * d
    for _ in range(3):
        y = y * (1.5 - half_d * y * y)
    return y


def _sc_disb_body(dst_hbm, out_hbm, idx_d, ones_v, stage_v, pk_v, dacc):
    c = lax.axis_index("c")
    s = lax.axis_index("s")

    def _fill(i, carry):
        stage_v[pl.ds(i * 16, 16)] = jnp.zeros((16,), jnp.float32)
        return carry
    lax.fori_loop(0, ZR // 16, _fill, 0)
    pltpu.sync_copy(stage_v, dacc.at[pl.ds(s * ZR, ZR)])

    def _ones(i, carry):
        ones_v[pl.ds(i * 16, 16)] = jnp.ones((16,), jnp.float32)
        return carry
    lax.fori_loop(0, 128 // 16, _ones, 0)
    pltpu.sync_copy(dst_hbm.at[2 * s], idx_d.at[pl.ds(0, NK)])
    pltpu.sync_copy(dst_hbm.at[2 * s + 1], idx_d.at[pl.ds(NK, NK)])
    plsc.subcore_barrier()

    def _chunk(j, carry):
        pltpu.sync_copy(ones_v.at[pl.ds(0, CH)], dacc.at[idx_d.at[j]],
                        add=True)
        return carry
    lax.fori_loop(0, 2 * NK, _chunk, 0)
    plsc.subcore_barrier()

    base = pl.multiple_of(c * (NT // 2) + s * ZR2, 8)
    pltpu.sync_copy(dacc.at[pl.ds(base, ZR2)], stage_v.at[pl.ds(0, ZR2)])

    def _rows(v, carry):
        d = stage_v[pl.ds(v * 16, 16)] + 1.0
        y = _newton_rsqrt(d)
        for l in range(16):
            pk_v[v * 16 + l, :] = jnp.broadcast_to(y[l], (16,))
        return carry
    lax.fori_loop(0, ZR2 // 16, _rows, 0)
    pltpu.sync_copy(pk_v, out_hbm.at[pl.ds(base, ZR2)])


_sc_disb_scratch = [
    pltpu.VMEM((2 * NK, CH), jnp.int32),
    pltpu.VMEM((128,), jnp.float32),
    pltpu.VMEM((ZR,), jnp.float32),
    pltpu.VMEM((ZR2, H), jnp.float32),
    pltpu.VMEM_SHARED((NT,), jnp.float32),
]

_sc_disb = pl.kernel(
    _sc_disb_body,
    out_type=jax.ShapeDtypeStruct((NT, H), jnp.float32),
    mesh=_mesh,
    scratch_types=_sc_disb_scratch,
    compiler_params=pltpu.CompilerParams(use_tc_tiling_on_sc=False, needs_layout_passes=False),
)


def _sc_scatter_body(q_hbm, src_hbm, dst_hbm, out0, out1,
                     idx_s, idx_d, rows, stage_v, zacc, qsp, sems):
    c = lax.axis_index("c")
    s = lax.axis_index("s")
    wid = s * NC + c

    def _fill(i, carry):
        stage_v[i, :] = jnp.zeros((16,), jnp.float32)
        return carry
    lax.fori_loop(0, ZR, _fill, 0)
    pltpu.sync_copy(q_hbm.at[pl.ds(s * ZR, ZR)], qsp.at[pl.ds(s * ZR, ZR)])
    pltpu.sync_copy(stage_v, zacc.at[pl.ds(s * ZR, ZR)])
    pltpu.sync_copy(src_hbm.at[wid], idx_s)
    pltpu.sync_copy(dst_hbm.at[wid], idx_d)
    plsc.subcore_barrier()

    def _group(g, carry):
        j0 = g * NB
        descs = [
            pltpu.async_copy(qsp.at[idx_s.at[j0 + b]], rows.at[b],
                             sems.at[b])
            for b in range(NB)
        ]
        for b in range(NB):
            descs[b].wait()
            pltpu.sync_copy(rows.at[b], zacc.at[idx_d.at[j0 + b]], add=True)
        return carry
    lax.fori_loop(0, NK // NB, _group, 0)
    plsc.subcore_barrier()

    pltpu.sync_copy(zacc.at[pl.ds(s * ZR, ZR)], stage_v)

    @pl.when(c == 0)
    def _():
        pltpu.sync_copy(stage_v, out0.at[pl.ds(s * ZR, ZR)])

    @pl.when(c == 1)
    def _():
        pltpu.sync_copy(stage_v, out1.at[pl.ds(s * ZR, ZR)])


_sc_scatter_scratch = [
    pltpu.VMEM((NK, CH), jnp.int32),
    pltpu.VMEM((NK, CH), jnp.int32),
    pltpu.VMEM((NB, CH, H), jnp.float32),
    pltpu.VMEM((ZR, H), jnp.float32),
    pltpu.VMEM_SHARED((NT, H), jnp.float32),
    pltpu.VMEM_SHARED((NT, H), jnp.float32),
    pltpu.SemaphoreType.DMA((NB,)),
]

_sc_scatter = pl.kernel(
    _sc_scatter_body,
    out_type=(jax.ShapeDtypeStruct((NT, H), jnp.float32),
              jax.ShapeDtypeStruct((NT, H), jnp.float32)),
    mesh=_mesh,
    scratch_types=_sc_scatter_scratch,
    compiler_params=pltpu.CompilerParams(use_tc_tiling_on_sc=False, needs_layout_passes=False),
)


RX = 1280
PX = RX // 8
RB8 = P8 // 8


def _tc0_body(x_ref, w_ref, p_ref):
    acc = jnp.zeros((PX, 128), jnp.float32)
    for n in range(8):
        acc += jnp.dot(x_ref[:, n, :], w_ref[n],
                       preferred_element_type=jnp.float32)
    p_ref[...] = acc


_tc0 = pl.pallas_call(
    _tc0_body,
    grid=(NT // RX,),
    in_specs=[
        pl.BlockSpec((PX, 8, D), lambda i: (i, 0, 0)),
        pl.BlockSpec((8, D, 128), lambda i: (0, 0, 0)),
    ],
    out_specs=pl.BlockSpec((PX, 128), lambda i: (i, 0)),
    out_shape=jax.ShapeDtypeStruct((P8, 128), jnp.float32),
)


def _tc1_body(p_ref, disb_ref, q_ref):
    q_ref[...] = p_ref[...] * disb_ref[...]


_tc1 = pl.pallas_call(
    _tc1_body,
    grid=(P8 // RB8,),
    in_specs=[
        pl.BlockSpec((RB8, 128), lambda i: (i, 0)),
        pl.BlockSpec((RB8, 128), lambda i: (i, 0)),
    ],
    out_specs=pl.BlockSpec((RB8, 128), lambda i: (i, 0)),
    out_shape=jax.ShapeDtypeStruct((P8, 128), jnp.float32),
)


def _tc2_body(z0_ref, z1_ref, q1_ref, disb_ref, b_ref, q2_ref):
    z = z0_ref[...] + z1_ref[...] + q1_ref[...]
    y = jnp.maximum(disb_ref[...] * z + b_ref[...], 0.0)
    q2_ref[...] = disb_ref[...] * y


_tc2 = pl.pallas_call(
    _tc2_body,
    grid=(P8 // RB8,),
    in_specs=[
        pl.BlockSpec((RB8, 128), lambda i: (i, 0)),
        pl.BlockSpec((RB8, 128), lambda i: (i, 0)),
        pl.BlockSpec((RB8, 128), lambda i: (i, 0)),
        pl.BlockSpec((RB8, 128), lambda i: (i, 0)),
        pl.BlockSpec((1, 128), lambda i: (0, 0)),
    ],
    out_specs=pl.BlockSpec((RB8, 128), lambda i: (i, 0)),
    out_shape=jax.ShapeDtypeStruct((P8, 128), jnp.float32),
)


def _tc2b_body(z0_ref, z1_ref, q2_ref, disb_ref, u_ref):
    u_ref[...] = disb_ref[...] * (z0_ref[...] + z1_ref[...] + q2_ref[...])


_tc2b = pl.pallas_call(
    _tc2b_body,
    grid=(P8 // RB8,),
    in_specs=[
        pl.BlockSpec((RB8, 128), lambda i: (i, 0)),
        pl.BlockSpec((RB8, 128), lambda i: (i, 0)),
        pl.BlockSpec((RB8, 128), lambda i: (i, 0)),
        pl.BlockSpec((RB8, 128), lambda i: (i, 0)),
    ],
    out_specs=pl.BlockSpec((RB8, 128), lambda i: (i, 0)),
    out_shape=jax.ShapeDtypeStruct((P8, 128), jnp.float32),
)


def _tc3_body(u_ref, w_ref, b_ref, o_ref):
    v = jnp.dot(u_ref[...], w_ref[...],
                preferred_element_type=jnp.float32) + b_ref[...]
    m = jnp.max(v, axis=1, keepdims=True)
    e = jnp.exp(v - m)
    lse = jnp.log(jnp.sum(e, axis=1, keepdims=True))
    o_ref[...] = v - m - lse


_tc3 = pl.pallas_call(
    _tc3_body,
    grid=(NT // RX,),
    in_specs=[
        pl.BlockSpec((RX, H), lambda i: (i, 0)),
        pl.BlockSpec((H, C), lambda i: (0, 0)),
        pl.BlockSpec((1, C), lambda i: (0, 0)),
    ],
    out_specs=pl.BlockSpec((RX, C), lambda i: (i, 0)),
    out_shape=jax.ShapeDtypeStruct((N, C), jnp.float32),
)


def kernel(x, edge_index, W1, b1, W2, b2):
    srcp = edge_index[0].reshape(NW, NK, CH)
    dstp = edge_index[1].reshape(NW, NK, CH)

    disb_l = _sc_disb(dstp)
    disb = disb_l.reshape(P8, 128)
    x3 = x.reshape(N // 8, 8, D)
    w1e = jnp.kron(jnp.eye(8, dtype=x.dtype), W1).reshape(8, D, 128)
    p = _tc0(x3, w1e)
    q1 = _tc1(p, disb)
    z0, z1 = _sc_scatter(q1.reshape(NT, H), srcp, dstp)
    q2 = _tc2(z0.reshape(P8, 128), z1.reshape(P8, 128), q1, disb,
              jnp.tile(b1, 8).reshape(1, 128))
    z0b, z1b = _sc_scatter(q2.reshape(NT, H), srcp, dstp)
    u = _tc2b(z0b.reshape(P8, 128), z1b.reshape(P8, 128), q2, disb)
    out = _tc3(u.reshape(NT, H), W2, b2.reshape(1, C))
    return out
